# baseline (speedup 1.0000x reference)
import functools

import jax
import jax.numpy as jnp
from jax import lax
from jax.experimental import pallas as pl
from jax.experimental.pallas import tpu as pltpu
from jax.experimental.pallas import tpu_sc as plsc

NC = 2
NS = 16
L = 16
NW = NC * NS
ROWS_OUT = 1024
ROWS_IN = 512
H = 12
E = 64
C = ROWS_OUT // NW
IDX_CHUNKS = ROWS_IN // L

_mesh = plsc.VectorSubcoreMesh(core_axis_name="c", subcore_axis_name="s")


@functools.partial(
    pl.kernel,
    out_type=jax.ShapeDtypeStruct((1, ROWS_OUT, H, E), jnp.float32),
    mesh=_mesh,
    scratch_types=[
        pltpu.VMEM((ROWS_IN,), jnp.int32),
        pltpu.VMEM((C, H, E), jnp.float32),
        pltpu.VMEM((C,), jnp.int32),
    ],
    compiler_params=pltpu.CompilerParams(use_tc_tiling_on_sc=False,
                                         needs_layout_passes=False),
)
def _index_put_sc(idx_hbm, kval_hbm, cache_hbm, out_hbm, idx_v, buf, slot_map):
    wid = lax.axis_index("s") * NC + lax.axis_index("c")
    base = wid * C

    pltpu.sync_copy(idx_hbm, idx_v)

    acc_lo = jnp.zeros((L,), jnp.int32)
    acc_in = jnp.zeros((L,), jnp.int32)
    one = jnp.ones((L,), jnp.int32)
    zero = jnp.zeros((L,), jnp.int32)
    for v in range(IDX_CHUNKS):
        vec = idx_v[pl.ds(v * L, L)]
        acc_lo = acc_lo + jnp.where(vec < base, one, zero)
        acc_in = acc_in + jnp.where((vec >= base) & (vec < base + C), one, zero)
    lo = jnp.sum(acc_lo)
    cnt = jnp.sum(acc_in)

    @pl.when(cnt == C)
    def _():
        pltpu.sync_copy(kval_hbm.at[0, pl.ds(lo, C)], buf)
        pltpu.sync_copy(buf, out_hbm.at[0, pl.ds(base, C)])

    @pl.when(cnt == 0)
    def _():
        pltpu.sync_copy(cache_hbm.at[0, pl.ds(base, C)], buf)
        pltpu.sync_copy(buf, out_hbm.at[0, pl.ds(base, C)])

    @pl.when((cnt > 0) & (cnt < C))
    def _():
        pltpu.sync_copy(cache_hbm.at[0, pl.ds(base, C)], buf)
        neg = jnp.full((L,), -1, jnp.int32)
        for s in range(C // L):
            slot_map[pl.ds(s * L, L)] = neg
        for v in range(IDX_CHUNKS):
            vec = idx_v[pl.ds(v * L, L)]
            rel = vec - base
            m = (rel >= 0) & (rel < C)
            src = lax.iota(jnp.int32, L) + (v * L)
            plsc.store_scatter(slot_map, [jnp.where(m, rel, 0)], src, mask=m)
        lanes = lax.iota(jnp.int32, L)
        for j in range(C):
            vec = slot_map[pl.ds((j // L) * L, L)]
            row = jnp.sum(jnp.where(lanes == (j % L), vec, zero))

            @pl.when(row >= 0)
            def _():
                pltpu.sync_copy(kval_hbm.at[0, pl.ds(row, 1)],
                                buf.at[pl.ds(j, 1)])

        pltpu.sync_copy(buf, out_hbm.at[0, pl.ds(base, C)])


def kernel(input_pos, k_val, k_cache):
    idx = input_pos.astype(jnp.int32)
    return _index_put_sc(idx, k_val, k_cache)


# speedup vs baseline: 1.3505x; 1.3505x over previous
import functools

import jax
import jax.numpy as jnp
from jax import lax
from jax.experimental import pallas as pl
from jax.experimental.pallas import tpu as pltpu
from jax.experimental.pallas import tpu_sc as plsc

NC = 2
NS = 16
L = 16
NW = NC * NS
ROWS_OUT = 1024
ROWS_IN = 512
H = 12
E = 64
C = ROWS_OUT // NW
IDX_CHUNKS = ROWS_IN // L

_mesh = plsc.VectorSubcoreMesh(core_axis_name="c", subcore_axis_name="s")


@functools.partial(
    pl.kernel,
    out_type=jax.ShapeDtypeStruct((1, ROWS_OUT, H, E), jnp.float32),
    mesh=_mesh,
    scratch_types=[
        pltpu.VMEM((ROWS_IN,), jnp.int32),
        pltpu.VMEM((C, H, E), jnp.float32),
        pltpu.VMEM((C,), jnp.int32),
    ],
    compiler_params=pltpu.CompilerParams(needs_layout_passes=False),
)
def _index_put_sc(idx_hbm, kval_hbm, cache_hbm, out_hbm, idx_v, buf, slot_map):
    wid = lax.axis_index("s") * NC + lax.axis_index("c")
    base = wid * C

    pltpu.sync_copy(idx_hbm, idx_v)

    acc_lo = jnp.zeros((L,), jnp.int32)
    acc_in = jnp.zeros((L,), jnp.int32)
    one = jnp.ones((L,), jnp.int32)
    zero = jnp.zeros((L,), jnp.int32)
    for v in range(IDX_CHUNKS):
        vec = idx_v[pl.ds(v * L, L)]
        acc_lo = acc_lo + jnp.where(vec < base, one, zero)
        acc_in = acc_in + jnp.where((vec >= base) & (vec < base + C), one, zero)
    lo = jnp.sum(acc_lo)
    cnt = jnp.sum(acc_in)

    @pl.when(cnt == C)
    def _():
        pltpu.sync_copy(kval_hbm.at[0, pl.ds(lo, C)], buf)
        pltpu.sync_copy(buf, out_hbm.at[0, pl.ds(base, C)])

    @pl.when(cnt == 0)
    def _():
        pltpu.sync_copy(cache_hbm.at[0, pl.ds(base, C)], buf)
        pltpu.sync_copy(buf, out_hbm.at[0, pl.ds(base, C)])

    @pl.when((cnt > 0) & (cnt < C))
    def _():
        pltpu.sync_copy(cache_hbm.at[0, pl.ds(base, C)], buf)
        neg = jnp.full((L,), -1, jnp.int32)
        for s in range(C // L):
            slot_map[pl.ds(s * L, L)] = neg
        for v in range(IDX_CHUNKS):
            vec = idx_v[pl.ds(v * L, L)]
            rel = vec - base
            m = (rel >= 0) & (rel < C)
            src = lax.iota(jnp.int32, L) + (v * L)
            plsc.store_scatter(slot_map, [jnp.where(m, rel, 0)], src, mask=m)
        lanes = lax.iota(jnp.int32, L)
        for j in range(C):
            vec = slot_map[pl.ds((j // L) * L, L)]
            row = jnp.sum(jnp.where(lanes == (j % L), vec, zero))

            @pl.when(row >= 0)
            def _():
                pltpu.sync_copy(kval_hbm.at[0, pl.ds(row, 1)],
                                buf.at[pl.ds(j, 1)])

        pltpu.sync_copy(buf, out_hbm.at[0, pl.ds(base, C)])


def kernel(input_pos, k_val, k_cache):
    idx = input_pos.astype(jnp.int32)
    return _index_put_sc(idx, k_val, k_cache)


# speedup vs baseline: 2.2828x; 1.6903x over previous
import functools

import jax
import jax.numpy as jnp
from jax import lax
from jax.experimental import pallas as pl
from jax.experimental.pallas import tpu as pltpu
from jax.experimental.pallas import tpu_sc as plsc

NC = 2
NS = 16
L = 16
NW = NC * NS
SEQ_OUT = 1024
SEQ_IN = 512
H = 12
E = 64
LINES = H * E
LB = LINES // NW
IDX_CHUNKS = SEQ_IN // L

_mesh = plsc.VectorSubcoreMesh(core_axis_name="c", subcore_axis_name="s")


@functools.partial(
    pl.kernel,
    out_type=jax.ShapeDtypeStruct((LINES, SEQ_OUT), jnp.float32),
    mesh=_mesh,
    scratch_types=[
        pltpu.VMEM((SEQ_IN,), jnp.int32),
        pltpu.VMEM((LB, SEQ_IN), jnp.float32),
        pltpu.VMEM((LB, SEQ_OUT), jnp.float32),
    ],
    compiler_params=pltpu.CompilerParams(needs_layout_passes=False),
)
def _index_put_sc(idx_hbm, kval_hbm, cache_hbm, out_hbm, idx_v, bufk, buff):
    wid = lax.axis_index("s") * NC + lax.axis_index("c")
    lb = pl.multiple_of(wid * LB, 8)

    pltpu.sync_copy(idx_hbm, idx_v)

    acc = jnp.zeros((L,), jnp.int32)
    one = jnp.ones((L,), jnp.int32)
    zero = jnp.zeros((L,), jnp.int32)
    lanes = lax.iota(jnp.int32, L)
    for v in range(IDX_CHUNKS):
        vec = idx_v[pl.ds(v * L, L)]
        acc = acc + jnp.where(vec == lanes + (v * L), one, zero)
    is_arange = jnp.sum(acc) == SEQ_IN

    @pl.when(is_arange)
    def _():
        pltpu.sync_copy(kval_hbm.at[pl.ds(lb, LB)], bufk)
        pltpu.sync_copy(bufk, out_hbm.at[pl.ds(lb, LB), pl.ds(0, SEQ_IN)])
        pltpu.sync_copy(cache_hbm.at[pl.ds(lb, LB), pl.ds(SEQ_IN, SEQ_IN)],
                        bufk)
        pltpu.sync_copy(bufk, out_hbm.at[pl.ds(lb, LB), pl.ds(SEQ_IN, SEQ_IN)])

    @pl.when(jnp.logical_not(is_arange))
    def _():
        pltpu.sync_copy(cache_hbm.at[pl.ds(lb, LB)], buff)
        pltpu.sync_copy(kval_hbm.at[pl.ds(lb, LB)], bufk)

        def body(r, carry):
            row = zero + r
            for c in range(IDX_CHUNKS):
                col = lanes + (c * L)
                pos = idx_v[pl.ds(c * L, L)]
                vals = plsc.load_gather(bufk, [row, col])
                plsc.store_scatter(buff, [row, pos], vals)
            return carry

        lax.fori_loop(0, LB, body, 0)
        pltpu.sync_copy(buff, out_hbm.at[pl.ds(lb, LB)])


def kernel(input_pos, k_val, k_cache):
    idx = input_pos.astype(jnp.int32)
    kv = jnp.transpose(k_val, (0, 2, 3, 1)).reshape(LINES, SEQ_IN)
    kc = jnp.transpose(k_cache, (0, 2, 3, 1)).reshape(LINES, SEQ_OUT)
    out = _index_put_sc(idx, kv, kc)
    return jnp.transpose(out.reshape(1, H, E, SEQ_OUT), (0, 3, 1, 2))


# speedup vs baseline: 2.4651x; 1.0798x over previous
import functools

import jax
import jax.numpy as jnp
from jax import lax
from jax.experimental import pallas as pl
from jax.experimental.pallas import tpu as pltpu
from jax.experimental.pallas import tpu_sc as plsc

NC = 2
NS = 16
L = 16
NW = NC * NS
SEQ_OUT = 1024
SEQ_IN = 512
H = 12
E = 64
LINES = H * E
LB = LINES // NW
IDX_CHUNKS = SEQ_IN // L

_mesh = plsc.VectorSubcoreMesh(core_axis_name="c", subcore_axis_name="s")


@functools.partial(
    pl.kernel,
    out_type=jax.ShapeDtypeStruct((LINES, SEQ_OUT), jnp.float32),
    mesh=_mesh,
    scratch_types=[
        pltpu.VMEM((SEQ_IN,), jnp.int32),
        pltpu.VMEM((LB, SEQ_IN), jnp.float32),
        pltpu.VMEM((LB, SEQ_OUT), jnp.float32),
        pltpu.SemaphoreType.DMA,
        pltpu.SemaphoreType.DMA,
        pltpu.SemaphoreType.DMA,
        pltpu.SemaphoreType.DMA,
    ],
    compiler_params=pltpu.CompilerParams(needs_layout_passes=False),
)
def _index_put_sc(idx_hbm, kval_hbm, cache_hbm, out_hbm,
                  idx_v, bufk, buff, sem_k, sem_c, sem_s1, sem_s2):
    wid = lax.axis_index("s") * NC + lax.axis_index("c")
    lb = pl.multiple_of(wid * LB, 8)

    ld_k = pltpu.async_copy(kval_hbm.at[pl.ds(lb, LB)], bufk, sem_k)
    ld_c = pltpu.async_copy(cache_hbm.at[pl.ds(lb, LB)], buff, sem_c)

    pltpu.sync_copy(idx_hbm, idx_v)

    lanes = lax.iota(jnp.int32, L)
    one = jnp.ones((L,), jnp.int32)
    zero = jnp.zeros((L,), jnp.int32)

    def cbody(v, a):
        off = pl.multiple_of(v * L, 8)
        vec = idx_v[pl.ds(off, L)]
        return a + jnp.where(vec == lanes + v * L, one, zero)

    acc = lax.fori_loop(0, IDX_CHUNKS, cbody, zero)
    is_arange = jnp.sum(acc) == SEQ_IN

    ld_k.wait()
    ld_c.wait()

    @pl.when(is_arange)
    def _():
        st1 = pltpu.async_copy(
            bufk, out_hbm.at[pl.ds(lb, LB), pl.ds(0, SEQ_IN)], sem_s1)
        st2 = pltpu.async_copy(
            buff.at[slice(None), pl.ds(SEQ_IN, SEQ_IN)],
            out_hbm.at[pl.ds(lb, LB), pl.ds(SEQ_IN, SEQ_IN)], sem_s2)
        st1.wait()
        st2.wait()

    @pl.when(jnp.logical_not(is_arange))
    def _():
        def body(r, carry):
            row = zero + r

            def sbody(c, cc):
                off = pl.multiple_of(c * L, 8)
                pos = idx_v[pl.ds(off, L)]
                col = lanes + c * L
                vals = plsc.load_gather(bufk, [row, col])
                plsc.store_scatter(buff, [row, pos], vals)
                return cc

            lax.fori_loop(0, IDX_CHUNKS, sbody, 0)
            return carry

        lax.fori_loop(0, LB, body, 0)
        pltpu.sync_copy(buff, out_hbm.at[pl.ds(lb, LB)])


def kernel(input_pos, k_val, k_cache):
    idx = input_pos.astype(jnp.int32)
    kv = jnp.transpose(k_val, (0, 2, 3, 1)).reshape(LINES, SEQ_IN)
    kc = jnp.transpose(k_cache, (0, 2, 3, 1)).reshape(LINES, SEQ_OUT)
    out = _index_put_sc(idx, kv, kc)
    return jnp.transpose(out.reshape(1, H, E, SEQ_OUT), (0, 3, 1, 2))
